# baseline (speedup 1.0000x reference)
import functools

import jax
import jax.numpy as jnp
from jax import lax
from jax.experimental import pallas as pl
from jax.experimental.pallas import tpu as pltpu
from jax.experimental.pallas import tpu_sc as plsc

VOCAB = 1000000
DIM = 64
WPR = DIM // 4
B_TOTAL = 4096 * 200

NUM_CORES = 2
NUM_SUBCORES = 16
NW = NUM_CORES * NUM_SUBCORES
PER_W = B_TOTAL // NW
K = 128
N_CHUNKS = PER_W // K

_mesh = plsc.VectorSubcoreMesh(core_axis_name="c", subcore_axis_name="s")


@functools.partial(
    pl.kernel,
    out_type=jax.ShapeDtypeStruct((B_TOTAL, DIM), jnp.float32),
    mesh=_mesh,
    scratch_types=[
        pltpu.VMEM_SHARED((VOCAB,), jnp.float32),
        pltpu.VMEM((K,), jnp.int32),
        pltpu.VMEM((K, WPR), jnp.int32),
        pltpu.VMEM((K,), jnp.float32),
        pltpu.VMEM((K, DIM), jnp.float32),
    ],
    compiler_params=pltpu.CompilerParams(use_tc_tiling_on_sc=False),
)
def _sc_dequant_gather(ids_hbm, qw_hbm, scale_hbm, out_hbm,
                       scale_sh, idx_v, rows_v, scl_v, out_v):
    cid = lax.axis_index("c")
    sid = lax.axis_index("s")
    wid = sid * NUM_CORES + cid
    base = wid * PER_W

    @pl.when(sid == 0)
    def _():
        pltpu.sync_copy(scale_hbm, scale_sh)

    plsc.subcore_barrier()

    lanes = lax.iota(jnp.int32, 16)
    word_sel = [(lanes >> 2) + 4 * q for q in range(4)]
    shl = (24 - 8 * (lanes & 3)).astype(jnp.int32)

    def chunk_body(i, carry):
        off = base + i * K
        pltpu.sync_copy(ids_hbm.at[pl.ds(off, K)], idx_v)
        pltpu.sync_copy(qw_hbm.at[idx_v], rows_v)
        pltpu.sync_copy(scale_sh.at[idx_v], scl_v)

        @plsc.parallel_loop(0, K // 16)
        def _grp(g):
            r0 = g * 16
            sv16 = scl_v[pl.ds(r0, 16)]
            for l in range(16):
                r = r0 + l
                w = rows_v[r, :]
                sv = sv16.at[jnp.full((16,), l, dtype=jnp.int32)].get(
                    mode="promise_in_bounds")
                for q in range(4):
                    gq = w.at[word_sel[q]].get(mode="promise_in_bounds")
                    b = (gq << shl) >> 24
                    out_v[r, pl.ds(16 * q, 16)] = b.astype(jnp.float32) * sv

        pltpu.sync_copy(out_v, out_hbm.at[pl.ds(off, K)])
        return carry

    lax.fori_loop(0, N_CHUNKS, chunk_body, 0)


def kernel(input_ids, q_weight, scale):
    ids_flat = input_ids.reshape(B_TOTAL)
    scale_flat = scale.reshape(VOCAB)
    qw_words = lax.bitcast_convert_type(
        q_weight.reshape(VOCAB, WPR, 4), jnp.int32)
    out = _sc_dequant_gather(ids_flat, qw_words, scale_flat)
    return out.reshape(input_ids.shape[0], input_ids.shape[1], DIM)


# speedup vs baseline: 1.1318x; 1.1318x over previous
import functools

import jax
import jax.numpy as jnp
from jax import lax
from jax.experimental import pallas as pl
from jax.experimental.pallas import tpu as pltpu
from jax.experimental.pallas import tpu_sc as plsc

VOCAB = 1000000
DIM = 64
WPR = DIM // 4
B_TOTAL = 4096 * 200

NUM_CORES = 2
NUM_SUBCORES = 16
NW = NUM_CORES * NUM_SUBCORES
PER_W = B_TOTAL // NW
K = 128
N_CHUNKS = PER_W // K

_mesh = plsc.VectorSubcoreMesh(core_axis_name="c", subcore_axis_name="s")


@functools.partial(
    pl.kernel,
    out_type=jax.ShapeDtypeStruct((B_TOTAL, DIM), jnp.float32),
    mesh=_mesh,
    scratch_types=[
        pltpu.VMEM_SHARED((VOCAB,), jnp.float32),
        pltpu.VMEM((K,), jnp.int32),
        pltpu.VMEM((K,), jnp.int32),
        pltpu.VMEM((K, WPR), jnp.int32),
        pltpu.VMEM((K, WPR), jnp.int32),
        pltpu.VMEM((K,), jnp.float32),
        pltpu.VMEM((K,), jnp.float32),
        pltpu.VMEM((K, DIM), jnp.float32),
        pltpu.VMEM((K, DIM), jnp.float32),
        pltpu.SemaphoreType.DMA,
        pltpu.SemaphoreType.DMA,
        pltpu.SemaphoreType.DMA,
        pltpu.SemaphoreType.DMA,
        pltpu.SemaphoreType.DMA,
        pltpu.SemaphoreType.DMA,
        pltpu.SemaphoreType.DMA,
        pltpu.SemaphoreType.DMA,
    ],
    compiler_params=pltpu.CompilerParams(use_tc_tiling_on_sc=False),
)
def _sc_dequant_gather(ids_hbm, qw_hbm, scale_hbm, out_hbm,
                       scale_sh,
                       idx_v0, idx_v1, rows_v0, rows_v1,
                       scl_v0, scl_v1, out_v0, out_v1,
                       sem_i0, sem_i1, sem_g0, sem_g1,
                       sem_s0, sem_s1, sem_w0, sem_w1):
    cid = lax.axis_index("c")
    sid = lax.axis_index("s")
    wid = sid * NUM_CORES + cid
    base = wid * PER_W

    @pl.when(sid == 0)
    def _():
        pltpu.sync_copy(scale_hbm, scale_sh)

    plsc.subcore_barrier()

    lanes = lax.iota(jnp.int32, 16)
    word_sel = [(lanes >> 2) + 4 * q for q in range(4)]
    shl = (24 - 8 * (lanes & 3)).astype(jnp.int32)

    idx_v = (idx_v0, idx_v1)
    rows_v = (rows_v0, rows_v1)
    scl_v = (scl_v0, scl_v1)
    out_v = (out_v0, out_v1)
    sem_i = (sem_i0, sem_i1)
    sem_g = (sem_g0, sem_g1)
    sem_s = (sem_s0, sem_s1)
    sem_w = (sem_w0, sem_w1)

    def issue_idx(c, b):
        pltpu.async_copy(ids_hbm.at[pl.ds(base + c * K, K)], idx_v[b],
                         sem_i[b])

    def wait_idx(b):
        pltpu.make_async_copy(ids_hbm.at[pl.ds(base, K)], idx_v[b],
                              sem_i[b]).wait()

    def issue_gathers(b):
        pltpu.async_copy(qw_hbm.at[idx_v[b]], rows_v[b], sem_g[b])
        pltpu.async_copy(scale_sh.at[idx_v[b]], scl_v[b], sem_s[b])

    def wait_gathers(b):
        pltpu.make_async_copy(qw_hbm.at[idx_v[b]], rows_v[b], sem_g[b]).wait()
        pltpu.make_async_copy(scale_sh.at[idx_v[b]], scl_v[b],
                              sem_s[b]).wait()

    def issue_write(c, b):
        pltpu.async_copy(out_v[b], out_hbm.at[pl.ds(base + c * K, K)],
                         sem_w[b])

    def wait_write(b):
        pltpu.make_async_copy(out_v[b], out_hbm.at[pl.ds(base, K)],
                              sem_w[b]).wait()

    def compute(b):
        @plsc.parallel_loop(0, K // 16)
        def _grp(g):
            r0 = g * 16
            sv16 = scl_v[b][pl.ds(r0, 16)]
            for l in range(16):
                r = r0 + l
                w = rows_v[b][r, :]
                sv = sv16.at[jnp.full((16,), l, dtype=jnp.int32)].get(
                    mode="promise_in_bounds")
                for q in range(4):
                    gq = w.at[word_sel[q]].get(mode="promise_in_bounds")
                    b_ = (gq << shl) >> 24
                    out_v[b][r, pl.ds(16 * q, 16)] = (
                        b_.astype(jnp.float32) * sv)

    N2 = N_CHUNKS // 2

    pltpu.sync_copy(ids_hbm.at[pl.ds(base, K)], idx_v0)
    issue_gathers(0)
    pltpu.sync_copy(ids_hbm.at[pl.ds(base + K, K)], idx_v1)

    def pair_body(i2, carry):
        c0 = 2 * i2
        not_last = i2 < N2 - 1

        issue_gathers(1)
        wait_gathers(0)

        @pl.when(not_last)
        def _():
            issue_idx(c0 + 2, 0)

        @pl.when(i2 >= 1)
        def _():
            wait_write(0)
        compute(0)
        issue_write(c0, 0)

        wait_gathers(1)

        @pl.when(not_last)
        def _():
            issue_idx(c0 + 3, 1)

        @pl.when(i2 >= 1)
        def _():
            wait_write(1)
        compute(1)
        issue_write(c0 + 1, 1)

        @pl.when(not_last)
        def _():
            wait_idx(0)
            issue_gathers(0)
            wait_idx(1)

        return carry

    lax.fori_loop(0, N2, pair_body, 0)
    wait_write(0)
    wait_write(1)


def kernel(input_ids, q_weight, scale):
    ids_flat = input_ids.reshape(B_TOTAL)
    scale_flat = scale.reshape(VOCAB)
    qw_words = lax.bitcast_convert_type(
        q_weight.reshape(VOCAB, WPR, 4), jnp.int32)
    out = _sc_dequant_gather(ids_flat, qw_words, scale_flat)
    return out.reshape(input_ids.shape[0], input_ids.shape[1], DIM)


# speedup vs baseline: 1.1542x; 1.0198x over previous
import functools

import jax
import jax.numpy as jnp
from jax import lax
from jax.experimental import pallas as pl
from jax.experimental.pallas import tpu as pltpu
from jax.experimental.pallas import tpu_sc as plsc

VOCAB = 1000000
DIM = 64
WPR = DIM // 4
IDS_ROWS = 4096
IDS_COLS = 200
B_TOTAL = IDS_ROWS * IDS_COLS

NUM_CORES = 2
NUM_SUBCORES = 16
NW = NUM_CORES * NUM_SUBCORES
ROWS_PER_W = IDS_ROWS // NW
K = IDS_COLS
N_CHUNKS = ROWS_PER_W

_mesh = plsc.VectorSubcoreMesh(core_axis_name="c", subcore_axis_name="s")


@functools.partial(
    pl.kernel,
    out_type=jax.ShapeDtypeStruct((B_TOTAL, DIM), jnp.float32),
    mesh=_mesh,
    scratch_types=[
        pltpu.VMEM_SHARED((VOCAB,), jnp.float32),
        pltpu.VMEM((K,), jnp.int32),
        pltpu.VMEM((K,), jnp.int32),
        pltpu.VMEM((K, WPR), jnp.int32),
        pltpu.VMEM((K, WPR), jnp.int32),
        pltpu.VMEM((K + 8,), jnp.float32),
        pltpu.VMEM((K + 8,), jnp.float32),
        pltpu.VMEM((K, DIM), jnp.float32),
        pltpu.VMEM((K, DIM), jnp.float32),
        pltpu.SemaphoreType.DMA,
        pltpu.SemaphoreType.DMA,
        pltpu.SemaphoreType.DMA,
        pltpu.SemaphoreType.DMA,
        pltpu.SemaphoreType.DMA,
        pltpu.SemaphoreType.DMA,
        pltpu.SemaphoreType.DMA,
        pltpu.SemaphoreType.DMA,
    ],
    compiler_params=pltpu.CompilerParams(use_tc_tiling_on_sc=False),
)
def _sc_dequant_gather(ids_hbm, qw_hbm, scale_hbm, out_hbm,
                       scale_sh,
                       idx_v0, idx_v1, rows_v0, rows_v1,
                       scl_v0, scl_v1, out_v0, out_v1,
                       sem_i0, sem_i1, sem_g0, sem_g1,
                       sem_s0, sem_s1, sem_w0, sem_w1):
    cid = lax.axis_index("c")
    sid = lax.axis_index("s")
    wid = sid * NUM_CORES + cid
    row0 = wid * ROWS_PER_W

    @pl.when(sid == 0)
    def _():
        pltpu.sync_copy(scale_hbm, scale_sh)

    plsc.subcore_barrier()

    lanes = lax.iota(jnp.int32, 16)
    word_sel = [(lanes >> 2) + 4 * q for q in range(4)]
    shl = (24 - 8 * (lanes & 3)).astype(jnp.int32)

    idx_v = (idx_v0, idx_v1)
    rows_v = (rows_v0, rows_v1)
    scl_v = (scl_v0, scl_v1)
    out_v = (out_v0, out_v1)
    sem_i = (sem_i0, sem_i1)
    sem_g = (sem_g0, sem_g1)
    sem_s = (sem_s0, sem_s1)
    sem_w = (sem_w0, sem_w1)

    def issue_idx(c, b):
        pltpu.async_copy(ids_hbm.at[row0 + c, :], idx_v[b], sem_i[b])

    def wait_idx(b):
        pltpu.make_async_copy(ids_hbm.at[0, :], idx_v[b], sem_i[b]).wait()

    def issue_gathers(b):
        pltpu.async_copy(qw_hbm.at[idx_v[b]], rows_v[b], sem_g[b])
        pltpu.async_copy(scale_sh.at[idx_v[b]], scl_v[b].at[pl.ds(0, K)],
                         sem_s[b])

    def wait_gathers(b):
        pltpu.make_async_copy(qw_hbm.at[idx_v[b]], rows_v[b], sem_g[b]).wait()
        pltpu.make_async_copy(scale_sh.at[idx_v[b]], scl_v[b].at[pl.ds(0, K)],
                              sem_s[b]).wait()

    def issue_write(c, b):
        pltpu.async_copy(out_v[b], out_hbm.at[pl.ds((row0 + c) * K, K)],
                         sem_w[b])

    def wait_write(b):
        pltpu.make_async_copy(out_v[b], out_hbm.at[pl.ds(0, K)],
                              sem_w[b]).wait()

    def compute(b):
        @plsc.parallel_loop(0, K // 8)
        def _grp(g):
            r0 = g * 8
            sv8 = scl_v[b][pl.ds(r0, 16)]
            for l in range(8):
                r = r0 + l
                w = rows_v[b][r, :]
                sv = sv8.at[jnp.full((16,), l, dtype=jnp.int32)].get(
                    mode="promise_in_bounds")
                for q in range(4):
                    gq = w.at[word_sel[q]].get(mode="promise_in_bounds")
                    b_ = (gq << shl) >> 24
                    out_v[b][r, pl.ds(16 * q, 16)] = (
                        b_.astype(jnp.float32) * sv)

    N2 = N_CHUNKS // 2

    pltpu.sync_copy(ids_hbm.at[row0, :], idx_v0)
    issue_gathers(0)
    pltpu.sync_copy(ids_hbm.at[row0 + 1, :], idx_v1)

    def pair_body(i2, carry):
        c0 = 2 * i2
        not_last = i2 < N2 - 1

        issue_gathers(1)
        wait_gathers(0)

        @pl.when(not_last)
        def _():
            issue_idx(c0 + 2, 0)

        @pl.when(i2 >= 1)
        def _():
            wait_write(0)
        compute(0)
        issue_write(c0, 0)

        wait_gathers(1)

        @pl.when(not_last)
        def _():
            issue_idx(c0 + 3, 1)

        @pl.when(i2 >= 1)
        def _():
            wait_write(1)
        compute(1)
        issue_write(c0 + 1, 1)

        @pl.when(not_last)
        def _():
            wait_idx(0)
            issue_gathers(0)
            wait_idx(1)

        return carry

    lax.fori_loop(0, N2, pair_body, 0)
    wait_write(0)
    wait_write(1)


def kernel(input_ids, q_weight, scale):
    scale_flat = scale.reshape(VOCAB)
    qw_words = lax.bitcast_convert_type(
        q_weight.reshape(VOCAB, WPR, 4), jnp.int32)
    out = _sc_dequant_gather(input_ids, qw_words, scale_flat)
    return out.reshape(IDS_ROWS, IDS_COLS, DIM)
